# Initial kernel scaffold; baseline (speedup 1.0000x reference)
import jax
import jax.numpy as jnp
from jax.experimental import pallas as pl


def kernel(features, edge_indexs, W0, att_src0, att_dst0, b0, W1, att_src1, att_dst1, b1):
    raise NotImplementedError("write your pallas kernel here")


# speedup vs baseline: 1.0712x; 1.0712x over previous
import functools

import jax
import jax.numpy as jnp
from jax.experimental import pallas as pl

N = 10000
D = 256
H = 8
C = D // H


def _mm_body(x_ref, w_ref, o_ref):
    o_ref[...] = jnp.dot(x_ref[...], w_ref[...], preferred_element_type=jnp.float32)


def _matmul(x, w):
    m, k = x.shape
    _, n = w.shape
    bm = 2000
    return pl.pallas_call(
        _mm_body,
        grid=(m // bm,),
        in_specs=[
            pl.BlockSpec((bm, k), lambda i: (i, 0)),
            pl.BlockSpec((k, n), lambda i: (0, 0)),
        ],
        out_specs=pl.BlockSpec((bm, n), lambda i: (i, 0)),
        out_shape=jax.ShapeDtypeStruct((m, n), jnp.float32),
    )(x, w)


def _gat_layer(x, src, dst, W, a_s, a_d, b):
    n = x.shape[0]
    h = _matmul(x, W).reshape(n, H, C)
    alpha_src = (h * a_s[None, :, :]).sum(-1)
    alpha_dst = (h * a_d[None, :, :]).sum(-1)
    e = jax.nn.leaky_relu(alpha_src[src] + alpha_dst[dst], negative_slope=0.2)
    ex = jnp.exp(e)
    den = jax.ops.segment_sum(ex, dst, num_segments=n)
    num = jax.ops.segment_sum(h[src] * ex[:, :, None], dst, num_segments=n)
    out = num / (den[:, :, None] + 1e-16)
    return out.reshape(n, H * C) + b


def kernel(features, edge_indexs, W0, att_src0, att_dst0, b0, W1, att_src1, att_dst1, b1):
    loop = jnp.arange(N, dtype=edge_indexs.dtype)
    src = jnp.concatenate([edge_indexs[0], loop])
    dst = jnp.concatenate([edge_indexs[1], loop])
    x = _gat_layer(features, src, dst, W0, att_src0, att_dst0, b0)
    x = _gat_layer(x, src, dst, W1, att_src1, att_dst1, b1)
    return jax.nn.gelu(x, approximate=True)


# speedup vs baseline: 30.3997x; 28.3791x over previous
import functools

import jax
import jax.numpy as jnp
from jax import lax
from jax.experimental import pallas as pl
from jax.experimental.pallas import tpu as pltpu
from jax.experimental.pallas import tpu_sc as plsc

N = 10000
D = 256
H = 8
C = D // H
E = 160000
EL = E + N
L = 16
NC = 2
NS = 16
NW = NC * NS
BLK = 128
NBT = -(-EL // (NS * BLK))
EP = NBT * NS * BLK
NBA = EP // (NW * BLK)
NPAD = 10240
DPAD = NPAD // 2
HD = D // NC
WR = EP // 8
BM = 2000


def _dense_tail(h, asm_ref, adm_ref, hst_ref, as_ref, ad_ref):
    z = jnp.zeros((h.shape[0], HD - L), jnp.float32)
    as2 = jnp.dot(h, asm_ref[...], preferred_element_type=jnp.float32)
    ad2 = jnp.dot(h, adm_ref[...], preferred_element_type=jnp.float32)
    hst_ref[0] = h[:, :HD]
    hst_ref[1] = h[:, HD:]
    as_ref[...] = jnp.concatenate([as2, z], axis=1)
    ad_ref[...] = jnp.concatenate([ad2, z], axis=1)


def _dense1_body(x_ref, w_ref, asm_ref, adm_ref, hst_ref, as_ref, ad_ref):
    h = jnp.dot(x_ref[...], w_ref[...], preferred_element_type=jnp.float32)
    _dense_tail(h, asm_ref, adm_ref, hst_ref, as_ref, ad_ref)


_DENSE_OUT_SPECS = [
    pl.BlockSpec((2, BM, HD), lambda i: (0, i, 0)),
    pl.BlockSpec((BM, HD), lambda i: (i, 0)),
    pl.BlockSpec((BM, HD), lambda i: (i, 0)),
]
_DENSE_OUT_SHAPE = [
    jax.ShapeDtypeStruct((2, N, HD), jnp.float32),
    jax.ShapeDtypeStruct((N, HD), jnp.float32),
    jax.ShapeDtypeStruct((N, HD), jnp.float32),
]


def _dense1(x, w, asm, adm):
    return pl.pallas_call(
        _dense1_body,
        grid=(N // BM,),
        in_specs=[
            pl.BlockSpec((BM, D), lambda i: (i, 0)),
            pl.BlockSpec((D, D), lambda i: (0, 0)),
            pl.BlockSpec((D, L), lambda i: (0, 0)),
            pl.BlockSpec((D, L), lambda i: (0, 0)),
        ],
        out_specs=_DENSE_OUT_SPECS,
        out_shape=_DENSE_OUT_SHAPE,
    )(x, w, asm, adm)


def _dense2_body(n0_ref, n1_ref, den_ref, e16_ref, b_ref, w_ref, asm_ref,
                 adm_ref, hst_ref, as_ref, ad_ref):
    dinv = 1.0 / den_ref[...]
    expand = jnp.dot(dinv, e16_ref[...], preferred_element_type=jnp.float32)
    x = jnp.concatenate([n0_ref[...], n1_ref[...]], axis=1) * expand + b_ref[...]
    h = jnp.dot(x, w_ref[...], preferred_element_type=jnp.float32)
    _dense_tail(h, asm_ref, adm_ref, hst_ref, as_ref, ad_ref)


def _dense2(num, den, e16, b, w, asm, adm):
    nb = N // BM
    return pl.pallas_call(
        _dense2_body,
        grid=(nb,),
        in_specs=[
            pl.BlockSpec((BM, HD), lambda i: (i, 0)),
            pl.BlockSpec((BM, HD), lambda i: (i + nb, 0)),
            pl.BlockSpec((BM, L), lambda i: (i, 0)),
            pl.BlockSpec((L, D), lambda i: (0, 0)),
            pl.BlockSpec((1, D), lambda i: (0, 0)),
            pl.BlockSpec((D, D), lambda i: (0, 0)),
            pl.BlockSpec((D, L), lambda i: (0, 0)),
            pl.BlockSpec((D, L), lambda i: (0, 0)),
        ],
        out_specs=_DENSE_OUT_SPECS,
        out_shape=_DENSE_OUT_SHAPE,
    )(num, num, den, e16, b, w, asm, adm)


def _final_body(n0_ref, n1_ref, den_ref, e16_ref, b_ref, o_ref):
    dinv = 1.0 / den_ref[...]
    expand = jnp.dot(dinv, e16_ref[...], preferred_element_type=jnp.float32)
    x = jnp.concatenate([n0_ref[...], n1_ref[...]], axis=1) * expand + b_ref[...]
    o_ref[...] = jax.nn.gelu(x, approximate=True)


def _final(num, den, e16, b):
    nb = N // BM
    return pl.pallas_call(
        _final_body,
        grid=(nb,),
        in_specs=[
            pl.BlockSpec((BM, HD), lambda i: (i, 0)),
            pl.BlockSpec((BM, HD), lambda i: (i + nb, 0)),
            pl.BlockSpec((BM, L), lambda i: (i, 0)),
            pl.BlockSpec((L, D), lambda i: (0, 0)),
            pl.BlockSpec((1, D), lambda i: (0, 0)),
        ],
        out_specs=pl.BlockSpec((BM, D), lambda i: (i, 0)),
        out_shape=jax.ShapeDtypeStruct((N, D), jnp.float32),
    )(num, num, den, e16, b)


_GDN = lax.GatherDimensionNumbers(
    offset_dims=(), collapsed_slice_dims=(0,), start_index_map=(0,))


def _lane_splat(v, lane):
    idx = jnp.full((L, 1), lane, jnp.int32)
    return lax.gather(v, idx, dimension_numbers=_GDN, slice_sizes=(1,),
                      mode=lax.GatherScatterMode.PROMISE_IN_BOUNDS)


_mesh = plsc.VectorSubcoreMesh(core_axis_name="c", subcore_axis_name="s")


@functools.partial(
    pl.kernel,
    out_type=(
        jax.ShapeDtypeStruct((WR, HD), jnp.float32),
        jax.ShapeDtypeStruct((2 * DPAD, HD), jnp.float32),
    ),
    mesh=_mesh,
    scratch_types=[
        pltpu.VMEM((NBA, BLK), jnp.int32),
        pltpu.VMEM((NBA, BLK), jnp.int32),
        pltpu.VMEM((NBA, BLK), jnp.int32),
        pltpu.VMEM((BLK, HD), jnp.float32),
        pltpu.VMEM((BLK, HD), jnp.float32),
        pltpu.VMEM((BLK, HD), jnp.float32),
        pltpu.VMEM((BLK // 8, HD), jnp.float32),
        pltpu.VMEM_SHARED((DPAD, HD), jnp.float32),
        pltpu.SemaphoreType.DMA,
        pltpu.SemaphoreType.DMA,
    ],
)
def _att_kernel(asx, adx, srcb, dstb, w_out, den_out,
                src_v, dst_v, gdst_v, asv, adv, denrow, wpack,
                den_sh, sem_a, sem_b):
    cid = lax.axis_index("c")
    sid = lax.axis_index("s")
    wid = cid * NS + sid

    pltpu.sync_copy(srcb.at[wid], src_v)
    pltpu.sync_copy(dstb.at[wid], dst_v)

    zv = jnp.zeros((L,), jnp.float32)

    def pre_body(i, carry):
        for j in range(BLK // L):
            s = pl.ds(j * L, L)
            gdst_v[i, s] = dst_v[i, s] >> 1
        return carry

    lax.fori_loop(0, NBA, pre_body, 0)

    def zbody(i, carry):
        for j in range(HD // L):
            denrow[i, pl.ds(j * L, L)] = zv
        return carry

    lax.fori_loop(0, BLK, zbody, 0)
    dzbase = sid * (DPAD // NS)
    for q in range(DPAD // NS // BLK):
        pltpu.sync_copy(denrow, den_sh.at[pl.ds(dzbase + q * BLK, BLK)])
    drem = DPAD // NS - (DPAD // NS // BLK) * BLK
    if drem:
        pltpu.sync_copy(
            denrow.at[pl.ds(0, drem)],
            den_sh.at[pl.ds(dzbase + (DPAD // NS // BLK) * BLK, drem)])
    plsc.subcore_barrier()

    def blk_body(blk, carry):
        ca = pltpu.async_copy(asx.at[src_v.at[blk]], asv, sem_a)
        cb = pltpu.async_copy(adx.at[dst_v.at[blk]], adv, sem_b)
        ca.wait()
        cb.wait()

        def sk(g, c2):
            dvec = dst_v[blk, pl.ds(g * L, L)]
            for k2 in range(L):
                k = g * L + k2
                t = asv[k, pl.ds(0, L)] + adv[k, pl.ds(0, L)]
                w = jnp.exp(jnp.maximum(t, 0.2 * t))
                wpack[g * 2 + k2 // 8, pl.ds((k2 % 8) * L, L)] = w
                dsp = _lane_splat(dvec, k2)
                par = (dsp & 1).astype(jnp.float32)
                denrow[k, pl.ds(0, L)] = w * (1.0 - par)
                denrow[k, pl.ds(L, L)] = w * par
            return c2

        lax.fori_loop(0, BLK // L, sk, 0)
        pltpu.sync_copy(wpack, w_out.at[pl.ds((wid * NBA + blk) * (BLK // 8),
                                              BLK // 8)])
        pltpu.sync_copy(denrow, den_sh.at[gdst_v.at[blk]], add=True)
        return carry

    lax.fori_loop(0, NBA, blk_body, 0)
    plsc.subcore_barrier()

    dchunk = DPAD // NS
    pltpu.sync_copy(den_sh.at[pl.ds(sid * dchunk, dchunk)],
                    den_out.at[pl.ds(cid * DPAD + sid * dchunk, dchunk)])


@functools.partial(
    pl.kernel,
    out_type=jax.ShapeDtypeStruct((2 * N, HD), jnp.float32),
    mesh=_mesh,
    scratch_types=[
        pltpu.VMEM((NBT, BLK), jnp.int32),
        pltpu.VMEM((NBT, BLK), jnp.int32),
        pltpu.VMEM((BLK, HD), jnp.float32),
        pltpu.VMEM((BLK // 8, HD), jnp.float32),
        pltpu.VMEM_SHARED((NPAD, HD), jnp.float32),
        pltpu.SemaphoreType.DMA,
        pltpu.SemaphoreType.DMA,
    ],
)
def _num_kernel(hst, w_in, srcb, dstb, num_out,
                gsrc_v, dst_v, hv, wv, num_sh, sem_w, sem_h):
    cid = lax.axis_index("c")
    sid = lax.axis_index("s")

    pltpu.sync_copy(srcb.at[sid], gsrc_v)
    pltpu.sync_copy(dstb.at[sid], dst_v)

    off = cid * N
    zv = jnp.zeros((L,), jnp.float32)

    def pre_body(i, carry):
        for j in range(BLK // L):
            s = pl.ds(j * L, L)
            gsrc_v[i, s] = gsrc_v[i, s] + off
        return carry

    lax.fori_loop(0, NBT, pre_body, 0)

    def zbody(i, carry):
        for j in range(HD // L):
            hv[i, pl.ds(j * L, L)] = zv
        return carry

    lax.fori_loop(0, BLK, zbody, 0)
    zbase = sid * (NPAD // NS)
    for q in range(NPAD // NS // BLK):
        pltpu.sync_copy(hv, num_sh.at[pl.ds(zbase + q * BLK, BLK)])
    plsc.subcore_barrier()

    hb = 4 * cid

    def blk_body(blk, carry):
        ch = pltpu.async_copy(hst.at[gsrc_v.at[blk]], hv, sem_h)
        cw = pltpu.async_copy(
            w_in.at[pl.ds((sid * NBT + blk) * (BLK // 8), BLK // 8)],
            wv, sem_w)
        ch.wait()
        cw.wait()

        def sk(g, c2):
            for k2 in range(L):
                k = g * L + k2
                w = wv[g * 2 + k2 // 8, pl.ds((k2 % 8) * L, L)]
                s = [_lane_splat(w, hb + i) for i in range(4)]
                for j in range(HD // L):
                    sl = pl.ds(j * L, L)
                    hv[k, sl] = hv[k, sl] * s[j // 2]
            return c2

        lax.fori_loop(0, BLK // L, sk, 0)
        pltpu.sync_copy(hv, num_sh.at[dst_v.at[blk]], add=True)
        return carry

    lax.fori_loop(0, NBT, blk_body, 0)
    plsc.subcore_barrier()

    ochunk = 624
    obase = sid * ochunk
    pltpu.sync_copy(num_sh.at[pl.ds(obase, ochunk)],
                    num_out.at[pl.ds(off + obase, ochunk)])
    tail_base = NS * ochunk
    tail = N - tail_base

    @pl.when(sid == 0)
    def _():
        pltpu.sync_copy(num_sh.at[pl.ds(tail_base, tail)],
                        num_out.at[pl.ds(off + tail_base, tail)])


def _att_mat(a):
    eye = jnp.eye(H, dtype=jnp.float32)
    m = (eye[:, None, :] * a[:, :, None]).reshape(D, H)
    return jnp.concatenate([m, m], axis=1)


def _pad_rows(x):
    return jnp.concatenate(
        [x, jnp.zeros((NPAD - N, HD), jnp.float32)], axis=0)


def _unpack_den(denp):
    d = denp.reshape(2, DPAD, HD // L, L)[:, :, :2, :].sum(0)
    return d.reshape(NPAD, L)[:N]


def _edge_phase(hst, asx, adx, srca, dsta, srcb, dstb):
    w_pk, denp = _att_kernel(_pad_rows(asx), _pad_rows(adx), srca, dsta)
    num = _num_kernel(hst.reshape(2 * N, HD), w_pk, srcb, dstb)
    return num, _unpack_den(denp)


def kernel(features, edge_indexs, W0, att_src0, att_dst0, b0,
           W1, att_src1, att_dst1, b1):
    loop = jnp.arange(N, dtype=jnp.int32)
    pad = EP - EL
    src = jnp.concatenate([edge_indexs[0].astype(jnp.int32), loop,
                           jnp.zeros((pad,), jnp.int32)])
    dst = jnp.concatenate([edge_indexs[1].astype(jnp.int32), loop,
                           jnp.full((pad,), N, jnp.int32)])
    srca = src.reshape(NW, NBA, BLK)
    dsta = dst.reshape(NW, NBA, BLK)
    srcb = src.reshape(NS, NBT, BLK)
    dstb = dst.reshape(NS, NBT, BLK)

    e16 = jnp.concatenate(
        [jnp.repeat(jnp.eye(H, dtype=jnp.float32), C, axis=1),
         jnp.zeros((H, D), jnp.float32)], axis=0)

    hst, asx, adx = _dense1(features, W0, _att_mat(att_src0), _att_mat(att_dst0))
    num, den = _edge_phase(hst, asx, adx, srca, dsta, srcb, dstb)

    hst2, asx2, adx2 = _dense2(num, den, e16, b0.reshape(1, D), W1,
                               _att_mat(att_src1), _att_mat(att_dst1))
    num2, den2 = _edge_phase(hst2, asx2, adx2, srca, dsta, srcb, dstb)

    return _final(num2, den2, e16, b1.reshape(1, D))


# speedup vs baseline: 38.5868x; 1.2693x over previous
import functools

import jax
import jax.numpy as jnp
from jax import lax
from jax.experimental import pallas as pl
from jax.experimental.pallas import tpu as pltpu
from jax.experimental.pallas import tpu_sc as plsc

N = 10000
D = 256
H = 8
C = D // H
E = 160000
EL = E + N
L = 16
NC = 2
NS = 16
NW = NC * NS
BLK = 128
NBT = -(-EL // (NS * BLK))
EP = NBT * NS * BLK
NBA = EP // (NW * BLK)
NR = EP // BLK
NPAD = 10240
DPAD = NPAD // 2
HD = D // NC
WR = EP // 8
BM = 2000


def _dense_tail(h, asm_ref, adm_ref, hst_ref, as_ref, ad_ref):
    z = jnp.zeros((h.shape[0], HD - L), jnp.float32)
    as2 = jnp.dot(h, asm_ref[...], preferred_element_type=jnp.float32)
    ad2 = jnp.dot(h, adm_ref[...], preferred_element_type=jnp.float32)
    hst_ref[0] = h[:, :HD]
    hst_ref[1] = h[:, HD:]
    as_ref[...] = jnp.concatenate([as2, z], axis=1)
    ad_ref[...] = jnp.concatenate([ad2, z], axis=1)


def _dense1_body(x_ref, w_ref, asm_ref, adm_ref, hst_ref, as_ref, ad_ref):
    h = jnp.dot(x_ref[...], w_ref[...], preferred_element_type=jnp.float32)
    _dense_tail(h, asm_ref, adm_ref, hst_ref, as_ref, ad_ref)


_DENSE_OUT_SPECS = [
    pl.BlockSpec((2, BM, HD), lambda i: (0, i, 0)),
    pl.BlockSpec((BM, HD), lambda i: (i, 0)),
    pl.BlockSpec((BM, HD), lambda i: (i, 0)),
]
_DENSE_OUT_SHAPE = [
    jax.ShapeDtypeStruct((2, N, HD), jnp.float32),
    jax.ShapeDtypeStruct((N, HD), jnp.float32),
    jax.ShapeDtypeStruct((N, HD), jnp.float32),
]


def _dense1(x, w, asm, adm):
    return pl.pallas_call(
        _dense1_body,
        grid=(N // BM,),
        in_specs=[
            pl.BlockSpec((BM, D), lambda i: (i, 0)),
            pl.BlockSpec((D, D), lambda i: (0, 0)),
            pl.BlockSpec((D, L), lambda i: (0, 0)),
            pl.BlockSpec((D, L), lambda i: (0, 0)),
        ],
        out_specs=_DENSE_OUT_SPECS,
        out_shape=_DENSE_OUT_SHAPE,
    )(x, w, asm, adm)


def _dense2_body(n0_ref, n1_ref, den_ref, e16_ref, b_ref, w_ref, asm_ref,
                 adm_ref, hst_ref, as_ref, ad_ref):
    dinv = 1.0 / den_ref[...]
    expand = jnp.dot(dinv, e16_ref[...], preferred_element_type=jnp.float32)
    x = jnp.concatenate([n0_ref[...], n1_ref[...]], axis=1) * expand + b_ref[...]
    h = jnp.dot(x, w_ref[...], preferred_element_type=jnp.float32)
    _dense_tail(h, asm_ref, adm_ref, hst_ref, as_ref, ad_ref)


def _dense2(num, den, e16, b, w, asm, adm):
    nb = N // BM
    return pl.pallas_call(
        _dense2_body,
        grid=(nb,),
        in_specs=[
            pl.BlockSpec((BM, HD), lambda i: (i, 0)),
            pl.BlockSpec((BM, HD), lambda i: (i + nb, 0)),
            pl.BlockSpec((BM, L), lambda i: (i, 0)),
            pl.BlockSpec((L, D), lambda i: (0, 0)),
            pl.BlockSpec((1, D), lambda i: (0, 0)),
            pl.BlockSpec((D, D), lambda i: (0, 0)),
            pl.BlockSpec((D, L), lambda i: (0, 0)),
            pl.BlockSpec((D, L), lambda i: (0, 0)),
        ],
        out_specs=_DENSE_OUT_SPECS,
        out_shape=_DENSE_OUT_SHAPE,
    )(num, num, den, e16, b, w, asm, adm)


def _final_body(n0_ref, n1_ref, den_ref, e16_ref, b_ref, o_ref):
    dinv = 1.0 / den_ref[...]
    expand = jnp.dot(dinv, e16_ref[...], preferred_element_type=jnp.float32)
    x = jnp.concatenate([n0_ref[...], n1_ref[...]], axis=1) * expand + b_ref[...]
    o_ref[...] = jax.nn.gelu(x, approximate=True)


def _final(num, den, e16, b):
    nb = N // BM
    return pl.pallas_call(
        _final_body,
        grid=(nb,),
        in_specs=[
            pl.BlockSpec((BM, HD), lambda i: (i, 0)),
            pl.BlockSpec((BM, HD), lambda i: (i + nb, 0)),
            pl.BlockSpec((BM, L), lambda i: (i, 0)),
            pl.BlockSpec((L, D), lambda i: (0, 0)),
            pl.BlockSpec((1, D), lambda i: (0, 0)),
        ],
        out_specs=pl.BlockSpec((BM, D), lambda i: (i, 0)),
        out_shape=jax.ShapeDtypeStruct((N, D), jnp.float32),
    )(num, num, den, e16, b)


_GDN = lax.GatherDimensionNumbers(
    offset_dims=(), collapsed_slice_dims=(0,), start_index_map=(0,))


def _lane_splat(v, lane):
    idx = jnp.full((L, 1), lane, jnp.int32)
    return lax.gather(v, idx, dimension_numbers=_GDN, slice_sizes=(1,),
                      mode=lax.GatherScatterMode.PROMISE_IN_BOUNDS)


_mesh = plsc.VectorSubcoreMesh(core_axis_name="c", subcore_axis_name="s")


@functools.partial(
    pl.kernel,
    out_type=(
        jax.ShapeDtypeStruct((WR, HD), jnp.float32),
        jax.ShapeDtypeStruct((2 * DPAD, HD), jnp.float32),
    ),
    mesh=_mesh,
    scratch_types=[
        pltpu.VMEM((3, BLK), jnp.int32),
        pltpu.VMEM((3, BLK), jnp.int32),
        pltpu.VMEM((BLK, HD), jnp.float32),
        pltpu.VMEM((BLK, HD), jnp.float32),
        pltpu.VMEM((BLK, HD), jnp.float32),
        pltpu.VMEM((BLK, HD), jnp.float32),
        pltpu.VMEM((BLK, HD), jnp.float32),
        pltpu.VMEM((BLK // 8, HD), jnp.float32),
        pltpu.VMEM_SHARED((DPAD, HD), jnp.float32),
        pltpu.SemaphoreType.DMA,
        pltpu.SemaphoreType.DMA,
        pltpu.SemaphoreType.DMA,
        pltpu.SemaphoreType.DMA,
        pltpu.SemaphoreType.DMA,
        pltpu.SemaphoreType.DMA,
    ],
)
def _att_kernel(asx, adx, sd, w_out, den_out,
                ix0, ix1, asv0, asv1, adv0, adv1, denrow, wpack,
                den_sh, si0, si1, sa0, sa1, sb0, sb1):
    cid = lax.axis_index("c")
    sid = lax.axis_index("s")
    wid = cid * NS + sid
    ix = (ix0, ix1)
    asv = (asv0, asv1)
    adv = (adv0, adv1)
    si = (si0, si1)
    sa = (sa0, sa1)
    sb = (sb0, sb1)
    rbase = wid * NBA

    zv = jnp.zeros((L,), jnp.float32)

    def zbody(i, carry):
        for j in range(HD // L):
            denrow[i, pl.ds(j * L, L)] = zv
        return carry

    lax.fori_loop(0, BLK, zbody, 0)
    dzbase = sid * (DPAD // NS)
    for q in range(DPAD // NS // BLK):
        pltpu.sync_copy(denrow, den_sh.at[pl.ds(dzbase + q * BLK, BLK)])
    drem = DPAD // NS - (DPAD // NS // BLK) * BLK
    if drem:
        pltpu.sync_copy(
            denrow.at[pl.ds(0, drem)],
            den_sh.at[pl.ds(dzbase + (DPAD // NS // BLK) * BLK, drem)])
    plsc.subcore_barrier()

    def issue_idx(blk, b):
        pltpu.async_copy(sd.at[rbase + blk], ix[b].at[pl.ds(0, 2)], si[b])

    def wait_idx_fix(blk, b):
        pltpu.make_async_copy(sd.at[rbase + blk], ix[b].at[pl.ds(0, 2)],
                              si[b]).wait()
        for j in range(BLK // L):
            s = pl.ds(j * L, L)
            ix[b][2, s] = ix[b][1, s] >> 1

    def issue_gather(b):
        pltpu.async_copy(asx.at[ix[b].at[0]], asv[b], sa[b])
        pltpu.async_copy(adx.at[ix[b].at[1]], adv[b], sb[b])

    def wait_gather(b):
        pltpu.make_async_copy(asx.at[ix[b].at[0]], asv[b], sa[b]).wait()
        pltpu.make_async_copy(adx.at[ix[b].at[1]], adv[b], sb[b]).wait()

    issue_idx(0, 0)
    wait_idx_fix(0, 0)
    issue_gather(0)
    issue_idx(1, 1)

    def blk_body(ii, carry):
        for b in range(2):
            blk = 2 * ii + b
            wait_gather(b)

            @pl.when(blk + 1 < NBA)
            def _():
                wait_idx_fix(blk + 1, 1 - b)
                issue_gather(1 - b)

            def sk(g, c2):
                dvec = ix[b][1, pl.ds(g * L, L)]
                for k2 in range(L):
                    k = g * L + k2
                    t = asv[b][k, pl.ds(0, L)] + adv[b][k, pl.ds(0, L)]
                    w = jnp.exp(jnp.maximum(t, 0.2 * t))
                    wpack[g * 2 + k2 // 8, pl.ds((k2 % 8) * L, L)] = w
                    dsp = _lane_splat(dvec, k2)
                    par = (dsp & 1).astype(jnp.float32)
                    denrow[k, pl.ds(0, L)] = w * (1.0 - par)
                    denrow[k, pl.ds(L, L)] = w * par
                return c2

            lax.fori_loop(0, BLK // L, sk, 0)
            pltpu.sync_copy(wpack,
                            w_out.at[pl.ds((rbase + blk) * (BLK // 8),
                                           BLK // 8)])
            pltpu.sync_copy(denrow, den_sh.at[ix[b].at[2]], add=True)

            @pl.when(blk + 2 < NBA)
            def _():
                issue_idx(blk + 2, b)

        return carry

    lax.fori_loop(0, NBA // 2, blk_body, 0)
    plsc.subcore_barrier()

    dchunk = DPAD // NS
    pltpu.sync_copy(den_sh.at[pl.ds(sid * dchunk, dchunk)],
                    den_out.at[pl.ds(cid * DPAD + sid * dchunk, dchunk)])


@functools.partial(
    pl.kernel,
    out_type=jax.ShapeDtypeStruct((2 * N, HD), jnp.float32),
    mesh=_mesh,
    scratch_types=[
        pltpu.VMEM((2, BLK), jnp.int32),
        pltpu.VMEM((2, BLK), jnp.int32),
        pltpu.VMEM((BLK, HD), jnp.float32),
        pltpu.VMEM((BLK, HD), jnp.float32),
        pltpu.VMEM((BLK // 8, HD), jnp.float32),
        pltpu.VMEM((BLK // 8, HD), jnp.float32),
        pltpu.VMEM_SHARED((NPAD, HD), jnp.float32),
        pltpu.SemaphoreType.DMA,
        pltpu.SemaphoreType.DMA,
        pltpu.SemaphoreType.DMA,
        pltpu.SemaphoreType.DMA,
        pltpu.SemaphoreType.DMA,
        pltpu.SemaphoreType.DMA,
    ],
)
def _num_kernel(hst, w_in, sd, num_out,
                ix0, ix1, hv0, hv1, wv0, wv1, num_sh,
                si0, si1, sh0, sh1, sw0, sw1):
    cid = lax.axis_index("c")
    sid = lax.axis_index("s")
    ix = (ix0, ix1)
    hv = (hv0, hv1)
    wv = (wv0, wv1)
    si = (si0, si1)
    sh = (sh0, sh1)
    sw = (sw0, sw1)
    rbase = sid * NBT
    off = cid * N

    zv = jnp.zeros((L,), jnp.float32)

    def zbody(i, carry):
        for j in range(HD // L):
            hv0[i, pl.ds(j * L, L)] = zv
        return carry

    lax.fori_loop(0, BLK, zbody, 0)
    zbase = sid * (NPAD // NS)
    for q in range(NPAD // NS // BLK):
        pltpu.sync_copy(hv0, num_sh.at[pl.ds(zbase + q * BLK, BLK)])
    plsc.subcore_barrier()

    hb = 4 * cid

    def wrows(blk):
        return w_in.at[pl.ds((rbase + blk) * (BLK // 8), BLK // 8)]

    def issue_idx(blk, b):
        pltpu.async_copy(sd.at[rbase + blk], ix[b], si[b])

    def wait_idx_fix(blk, b):
        pltpu.make_async_copy(sd.at[rbase + blk], ix[b], si[b]).wait()
        for j in range(BLK // L):
            s = pl.ds(j * L, L)
            ix[b][0, s] = ix[b][0, s] + off

    def issue_gather(blk, b):
        pltpu.async_copy(hst.at[ix[b].at[0]], hv[b], sh[b])
        pltpu.async_copy(wrows(blk), wv[b], sw[b])

    def wait_gather(blk, b):
        pltpu.make_async_copy(hst.at[ix[b].at[0]], hv[b], sh[b]).wait()
        pltpu.make_async_copy(wrows(blk), wv[b], sw[b]).wait()

    issue_idx(0, 0)
    wait_idx_fix(0, 0)
    issue_gather(0, 0)
    issue_idx(1, 1)

    def blk_body(ii, carry):
        for b in range(2):
            blk = 2 * ii + b
            wait_gather(blk, b)

            @pl.when(blk + 1 < NBT)
            def _():
                wait_idx_fix(blk + 1, 1 - b)
                issue_gather(blk + 1, 1 - b)

            def sk(g, c2):
                for k2 in range(L):
                    k = g * L + k2
                    w = wv[b][g * 2 + k2 // 8, pl.ds((k2 % 8) * L, L)]
                    s = [_lane_splat(w, hb + i) for i in range(4)]
                    for j in range(HD // L):
                        sl = pl.ds(j * L, L)
                        hv[b][k, sl] = hv[b][k, sl] * s[j // 2]
                return c2

            lax.fori_loop(0, BLK // L, sk, 0)
            pltpu.sync_copy(hv[b], num_sh.at[ix[b].at[1]], add=True)

            @pl.when(blk + 2 < NBT)
            def _():
                issue_idx(blk + 2, b)

        return carry

    lax.fori_loop(0, NBT // 2, blk_body, 0)
    plsc.subcore_barrier()

    ochunk = 624
    obase = sid * ochunk
    pltpu.sync_copy(num_sh.at[pl.ds(obase, ochunk)],
                    num_out.at[pl.ds(off + obase, ochunk)])
    tail_base = NS * ochunk
    tail = N - tail_base

    @pl.when(sid == 0)
    def _():
        pltpu.sync_copy(num_sh.at[pl.ds(tail_base, tail)],
                        num_out.at[pl.ds(off + tail_base, tail)])


def _att_mat(a):
    eye = jnp.eye(H, dtype=jnp.float32)
    m = (eye[:, None, :] * a[:, :, None]).reshape(D, H)
    return jnp.concatenate([m, m], axis=1)


def _pad_rows(x):
    return jnp.concatenate(
        [x, jnp.zeros((NPAD - N, HD), jnp.float32)], axis=0)


def _unpack_den(denp):
    d = denp.reshape(2, DPAD, HD // L, L)[:, :, :2, :].sum(0)
    return d.reshape(NPAD, L)[:N]


def _edge_phase(hst, asx, adx, sd):
    w_pk, denp = _att_kernel(_pad_rows(asx), _pad_rows(adx), sd)
    num = _num_kernel(hst.reshape(2 * N, HD), w_pk, sd)
    return num, _unpack_den(denp)


def kernel(features, edge_indexs, W0, att_src0, att_dst0, b0,
           W1, att_src1, att_dst1, b1):
    loop = jnp.arange(N, dtype=jnp.int32)
    pad = EP - EL
    src = jnp.concatenate([edge_indexs[0].astype(jnp.int32), loop,
                           jnp.zeros((pad,), jnp.int32)])
    dst = jnp.concatenate([edge_indexs[1].astype(jnp.int32), loop,
                           jnp.full((pad,), N, jnp.int32)])
    sd = jnp.stack([src.reshape(NR, BLK), dst.reshape(NR, BLK)], axis=1)

    e16 = jnp.concatenate(
        [jnp.repeat(jnp.eye(H, dtype=jnp.float32), C, axis=1),
         jnp.zeros((H, D), jnp.float32)], axis=0)

    hst, asx, adx = _dense1(features, W0, _att_mat(att_src0), _att_mat(att_dst0))
    num, den = _edge_phase(hst, asx, adx, sd)

    hst2, asx2, adx2 = _dense2(num, den, e16, b0.reshape(1, D), W1,
                               _att_mat(att_src1), _att_mat(att_dst1))
    num2, den2 = _edge_phase(hst2, asx2, adx2, sd)

    return _final(num2, den2, e16, b1.reshape(1, D))


# speedup vs baseline: 40.7070x; 1.0549x over previous
import functools

import jax
import jax.numpy as jnp
from jax import lax
from jax.experimental import pallas as pl
from jax.experimental.pallas import tpu as pltpu
from jax.experimental.pallas import tpu_sc as plsc

N = 10000
D = 256
H = 8
C = D // H
E = 160000
EL = E + N
L = 16
NC = 2
NS = 16
NW = NC * NS
BLK = 128
NBT = -(-EL // (NS * BLK))
EP = NBT * NS * BLK
NBA = EP // (NW * BLK)
NR = EP // BLK
NPAD = 10240
DPAD = NPAD // 2
HD = D // NC
WR = EP // 8
BM = 2000


def _dense_tail(h, asm_ref, adm_ref, hst_ref, as_ref, ad_ref):
    z = jnp.zeros((h.shape[0], HD - L), jnp.float32)
    as2 = jnp.dot(h, asm_ref[...], preferred_element_type=jnp.float32)
    ad2 = jnp.dot(h, adm_ref[...], preferred_element_type=jnp.float32)
    hst_ref[0] = h[:, :HD]
    hst_ref[1] = h[:, HD:]
    as_ref[...] = jnp.concatenate([as2, z], axis=1)
    ad_ref[...] = jnp.concatenate([ad2, z], axis=1)


def _dense1_body(x_ref, w_ref, asm_ref, adm_ref, hst_ref, as_ref, ad_ref):
    h = jnp.dot(x_ref[...], w_ref[...], preferred_element_type=jnp.float32)
    _dense_tail(h, asm_ref, adm_ref, hst_ref, as_ref, ad_ref)


_DENSE_OUT_SPECS = [
    pl.BlockSpec((2, BM, HD), lambda i: (0, i, 0)),
    pl.BlockSpec((BM, HD), lambda i: (i, 0)),
    pl.BlockSpec((BM, HD), lambda i: (i, 0)),
]
_DENSE_OUT_SHAPE = [
    jax.ShapeDtypeStruct((2, N, HD), jnp.float32),
    jax.ShapeDtypeStruct((N, HD), jnp.float32),
    jax.ShapeDtypeStruct((N, HD), jnp.float32),
]


def _dense1(x, w, asm, adm):
    return pl.pallas_call(
        _dense1_body,
        grid=(N // BM,),
        in_specs=[
            pl.BlockSpec((BM, D), lambda i: (i, 0)),
            pl.BlockSpec((D, D), lambda i: (0, 0)),
            pl.BlockSpec((D, L), lambda i: (0, 0)),
            pl.BlockSpec((D, L), lambda i: (0, 0)),
        ],
        out_specs=_DENSE_OUT_SPECS,
        out_shape=_DENSE_OUT_SHAPE,
    )(x, w, asm, adm)


def _dense2_body(n0_ref, n1_ref, den_ref, e16_ref, b_ref, w_ref, asm_ref,
                 adm_ref, hst_ref, as_ref, ad_ref):
    dinv = 1.0 / den_ref[...]
    expand = jnp.dot(dinv, e16_ref[...], preferred_element_type=jnp.float32)
    x = jnp.concatenate([n0_ref[...], n1_ref[...]], axis=1) * expand + b_ref[...]
    h = jnp.dot(x, w_ref[...], preferred_element_type=jnp.float32)
    _dense_tail(h, asm_ref, adm_ref, hst_ref, as_ref, ad_ref)


def _dense2(num, den, e16, b, w, asm, adm):
    nb = N // BM
    return pl.pallas_call(
        _dense2_body,
        grid=(nb,),
        in_specs=[
            pl.BlockSpec((BM, HD), lambda i: (i, 0)),
            pl.BlockSpec((BM, HD), lambda i: (i + nb, 0)),
            pl.BlockSpec((BM, L), lambda i: (i, 0)),
            pl.BlockSpec((L, D), lambda i: (0, 0)),
            pl.BlockSpec((1, D), lambda i: (0, 0)),
            pl.BlockSpec((D, D), lambda i: (0, 0)),
            pl.BlockSpec((D, L), lambda i: (0, 0)),
            pl.BlockSpec((D, L), lambda i: (0, 0)),
        ],
        out_specs=_DENSE_OUT_SPECS,
        out_shape=_DENSE_OUT_SHAPE,
    )(num, num, den, e16, b, w, asm, adm)


def _final_body(n0_ref, n1_ref, den_ref, e16_ref, b_ref, o_ref):
    dinv = 1.0 / den_ref[...]
    expand = jnp.dot(dinv, e16_ref[...], preferred_element_type=jnp.float32)
    x = jnp.concatenate([n0_ref[...], n1_ref[...]], axis=1) * expand + b_ref[...]
    o_ref[...] = jax.nn.gelu(x, approximate=True)


def _final(num, den, e16, b):
    nb = N // BM
    return pl.pallas_call(
        _final_body,
        grid=(nb,),
        in_specs=[
            pl.BlockSpec((BM, HD), lambda i: (i, 0)),
            pl.BlockSpec((BM, HD), lambda i: (i + nb, 0)),
            pl.BlockSpec((BM, L), lambda i: (i, 0)),
            pl.BlockSpec((L, D), lambda i: (0, 0)),
            pl.BlockSpec((1, D), lambda i: (0, 0)),
        ],
        out_specs=pl.BlockSpec((BM, D), lambda i: (i, 0)),
        out_shape=jax.ShapeDtypeStruct((N, D), jnp.float32),
    )(num, num, den, e16, b)


_GDN = lax.GatherDimensionNumbers(
    offset_dims=(), collapsed_slice_dims=(0,), start_index_map=(0,))


def _lane_splat(v, lane):
    idx = jnp.full((L, 1), lane, jnp.int32)
    return lax.gather(v, idx, dimension_numbers=_GDN, slice_sizes=(1,),
                      mode=lax.GatherScatterMode.PROMISE_IN_BOUNDS)


_mesh = plsc.VectorSubcoreMesh(core_axis_name="c", subcore_axis_name="s")


@functools.partial(
    pl.kernel,
    out_type=(
        jax.ShapeDtypeStruct((WR, HD), jnp.float32),
        jax.ShapeDtypeStruct((2 * DPAD, HD), jnp.float32),
    ),
    mesh=_mesh,
    scratch_types=[
        pltpu.VMEM((3, BLK), jnp.int32),
        pltpu.VMEM((3, BLK), jnp.int32),
        pltpu.VMEM((BLK, HD), jnp.float32),
        pltpu.VMEM((BLK, HD), jnp.float32),
        pltpu.VMEM((BLK, HD), jnp.float32),
        pltpu.VMEM((BLK, HD), jnp.float32),
        pltpu.VMEM((BLK, HD), jnp.float32),
        pltpu.VMEM((BLK // 8, HD), jnp.float32),
        pltpu.VMEM_SHARED((DPAD, HD), jnp.float32),
        pltpu.SemaphoreType.DMA,
        pltpu.SemaphoreType.DMA,
        pltpu.SemaphoreType.DMA,
        pltpu.SemaphoreType.DMA,
        pltpu.SemaphoreType.DMA,
        pltpu.SemaphoreType.DMA,
    ],
)
def _att_kernel(asx, adx, sd, w_out, den_out,
                ix0, ix1, asv0, asv1, adv0, adv1, denrow, wpack,
                den_sh, si0, si1, sa0, sa1, sb0, sb1):
    cid = lax.axis_index("c")
    sid = lax.axis_index("s")
    wid = cid * NS + sid
    ix = (ix0, ix1)
    asv = (asv0, asv1)
    adv = (adv0, adv1)
    si = (si0, si1)
    sa = (sa0, sa1)
    sb = (sb0, sb1)
    rbase = wid * NBA

    zv = jnp.zeros((L,), jnp.float32)

    def zbody(i, carry):
        for j in range(HD // L):
            denrow[i, pl.ds(j * L, L)] = zv
        return carry

    lax.fori_loop(0, BLK, zbody, 0)
    dzbase = sid * (DPAD // NS)
    for q in range(DPAD // NS // BLK):
        pltpu.sync_copy(denrow, den_sh.at[pl.ds(dzbase + q * BLK, BLK)])
    drem = DPAD // NS - (DPAD // NS // BLK) * BLK
    if drem:
        pltpu.sync_copy(
            denrow.at[pl.ds(0, drem)],
            den_sh.at[pl.ds(dzbase + (DPAD // NS // BLK) * BLK, drem)])
    plsc.subcore_barrier()

    def issue_idx(blk, b):
        pltpu.async_copy(sd.at[rbase + blk], ix[b].at[pl.ds(0, 2)], si[b])

    def wait_idx_fix(blk, b):
        pltpu.make_async_copy(sd.at[rbase + blk], ix[b].at[pl.ds(0, 2)],
                              si[b]).wait()
        for j in range(BLK // L):
            s = pl.ds(j * L, L)
            ix[b][2, s] = ix[b][1, s] >> 1

    def issue_gather(b):
        pltpu.async_copy(asx.at[ix[b].at[0]], asv[b], sa[b])
        pltpu.async_copy(adx.at[ix[b].at[1]], adv[b], sb[b])

    def wait_gather(b):
        pltpu.make_async_copy(asx.at[ix[b].at[0]], asv[b], sa[b]).wait()
        pltpu.make_async_copy(adx.at[ix[b].at[1]], adv[b], sb[b]).wait()

    issue_idx(0, 0)
    wait_idx_fix(0, 0)
    issue_gather(0)
    issue_idx(1, 1)

    def blk_body(ii, carry):
        for b in range(2):
            blk = 2 * ii + b
            wait_gather(b)

            @pl.when(blk + 1 < NBA)
            def _():
                wait_idx_fix(blk + 1, 1 - b)
                issue_gather(1 - b)

            def sk(g, c2):
                dvec = ix[b][1, pl.ds(g * L, L)]
                for k2 in range(L):
                    k = g * L + k2
                    t = asv[b][k, pl.ds(0, L)] + adv[b][k, pl.ds(0, L)]
                    w = jnp.exp(jnp.maximum(t, 0.2 * t))
                    wpack[g * 2 + k2 // 8, pl.ds((k2 % 8) * L, L)] = w
                    dsp = _lane_splat(dvec, k2)
                    par = (dsp & 1).astype(jnp.float32)
                    denrow[k, pl.ds(0, L)] = w * (1.0 - par)
                    denrow[k, pl.ds(L, L)] = w * par
                return c2

            lax.fori_loop(0, BLK // L, sk, 0)
            pltpu.sync_copy(wpack,
                            w_out.at[pl.ds((rbase + blk) * (BLK // 8),
                                           BLK // 8)])
            pltpu.sync_copy(denrow, den_sh.at[ix[b].at[2]], add=True)

            @pl.when(blk + 2 < NBA)
            def _():
                issue_idx(blk + 2, b)

        return carry

    lax.fori_loop(0, NBA // 2, blk_body, 0)
    plsc.subcore_barrier()

    dchunk = DPAD // NS
    pltpu.sync_copy(den_sh.at[pl.ds(sid * dchunk, dchunk)],
                    den_out.at[pl.ds(cid * DPAD + sid * dchunk, dchunk)])


@functools.partial(
    pl.kernel,
    out_type=jax.ShapeDtypeStruct((2 * N, HD), jnp.float32),
    mesh=_mesh,
    scratch_types=[
        pltpu.VMEM((2, BLK), jnp.int32),
        pltpu.VMEM((2, BLK), jnp.int32),
        pltpu.VMEM((1, BLK), jnp.int32),
        pltpu.VMEM((1, BLK), jnp.int32),
        pltpu.VMEM((BLK, HD), jnp.float32),
        pltpu.VMEM((BLK, HD), jnp.float32),
        pltpu.VMEM((BLK // 8, HD), jnp.float32),
        pltpu.VMEM((BLK // 8, HD), jnp.float32),
        pltpu.VMEM_SHARED((NPAD, HD), jnp.float32),
        pltpu.SemaphoreType.DMA,
        pltpu.SemaphoreType.DMA,
        pltpu.SemaphoreType.DMA,
        pltpu.SemaphoreType.DMA,
        pltpu.SemaphoreType.DMA,
        pltpu.SemaphoreType.DMA,
        pltpu.SemaphoreType.DMA,
        pltpu.SemaphoreType.DMA,
    ],
)
def _num_kernel(hst, w_in, sd, num_out,
                ix0, ix1, six0, six1, hv0, hv1, wv0, wv1, num_sh,
                si0, si1, sh0, sh1, sw0, sw1, ss0, ss1):
    cid = lax.axis_index("c")
    sid = lax.axis_index("s")
    ix = (ix0, ix1)
    six = (six0, six1)
    hv = (hv0, hv1)
    wv = (wv0, wv1)
    si = (si0, si1)
    sh = (sh0, sh1)
    sw = (sw0, sw1)
    ss = (ss0, ss1)
    rbase = sid * NBT
    off = cid * N

    zv = jnp.zeros((L,), jnp.float32)

    def zbody(i, carry):
        for j in range(HD // L):
            hv0[i, pl.ds(j * L, L)] = zv
        return carry

    lax.fori_loop(0, BLK, zbody, 0)
    zbase = sid * (NPAD // NS)
    for q in range(NPAD // NS // BLK):
        pltpu.sync_copy(hv0, num_sh.at[pl.ds(zbase + q * BLK, BLK)])
    plsc.subcore_barrier()

    hb = 4 * cid

    def wrows(blk):
        return w_in.at[pl.ds((rbase + blk) * (BLK // 8), BLK // 8)]

    def issue_idx(blk, b):
        pltpu.async_copy(sd.at[rbase + blk], ix[b], si[b])

    def wait_idx_fix(blk, b):
        pltpu.make_async_copy(sd.at[rbase + blk], ix[b], si[b]).wait()
        for j in range(BLK // L):
            s = pl.ds(j * L, L)
            ix[b][0, s] = ix[b][0, s] + off

    def issue_gather(blk, b):
        pltpu.async_copy(hst.at[ix[b].at[0]], hv[b], sh[b])
        pltpu.async_copy(wrows(blk), wv[b], sw[b])

    def wait_gather(blk, b):
        pltpu.make_async_copy(hst.at[ix[b].at[0]], hv[b], sh[b]).wait()
        pltpu.make_async_copy(wrows(blk), wv[b], sw[b]).wait()

    def wait_scatter(b):
        pltpu.make_async_copy(hv[b], num_sh.at[six[b].at[0]], ss[b]).wait()

    issue_idx(0, 0)
    wait_idx_fix(0, 0)
    issue_gather(0, 0)
    issue_idx(1, 1)

    def blk_body(ii, carry):
        for b in range(2):
            blk = 2 * ii + b
            wait_gather(blk, b)

            @pl.when(blk + 1 < NBT)
            def _():
                @pl.when(blk >= 1)
                def _():
                    wait_scatter(1 - b)

                wait_idx_fix(blk + 1, 1 - b)
                issue_gather(blk + 1, 1 - b)

            def sk(g, c2):
                for k2 in range(L):
                    k = g * L + k2
                    w = wv[b][g * 2 + k2 // 8, pl.ds((k2 % 8) * L, L)]
                    s = [_lane_splat(w, hb + i) for i in range(4)]
                    for j in range(HD // L):
                        sl = pl.ds(j * L, L)
                        hv[b][k, sl] = hv[b][k, sl] * s[j // 2]
                return c2

            lax.fori_loop(0, BLK // L, sk, 0)
            for j in range(BLK // L):
                s = pl.ds(j * L, L)
                six[b][0, s] = ix[b][1, s]
            pltpu.async_copy(hv[b], num_sh.at[six[b].at[0]], ss[b], add=True)

            @pl.when(blk + 2 < NBT)
            def _():
                issue_idx(blk + 2, b)

        return carry

    lax.fori_loop(0, NBT // 2, blk_body, 0)
    wait_scatter(0)
    wait_scatter(1)
    plsc.subcore_barrier()

    ochunk = 624
    obase = sid * ochunk
    pltpu.sync_copy(num_sh.at[pl.ds(obase, ochunk)],
                    num_out.at[pl.ds(off + obase, ochunk)])
    tail_base = NS * ochunk
    tail = N - tail_base

    @pl.when(sid == 0)
    def _():
        pltpu.sync_copy(num_sh.at[pl.ds(tail_base, tail)],
                        num_out.at[pl.ds(off + tail_base, tail)])


def _att_mat(a):
    eye = jnp.eye(H, dtype=jnp.float32)
    m = (eye[:, None, :] * a[:, :, None]).reshape(D, H)
    return jnp.concatenate([m, m], axis=1)


def _pad_rows(x):
    return jnp.concatenate(
        [x, jnp.zeros((NPAD - N, HD), jnp.float32)], axis=0)


def _unpack_den(denp):
    d = denp.reshape(2, DPAD, HD // L, L)[:, :, :2, :].sum(0)
    return d.reshape(NPAD, L)[:N]


def _edge_phase(hst, asx, adx, sd):
    w_pk, denp = _att_kernel(_pad_rows(asx), _pad_rows(adx), sd)
    num = _num_kernel(hst.reshape(2 * N, HD), w_pk, sd)
    return num, _unpack_den(denp)


def kernel(features, edge_indexs, W0, att_src0, att_dst0, b0,
           W1, att_src1, att_dst1, b1):
    loop = jnp.arange(N, dtype=jnp.int32)
    pad = EP - EL
    src = jnp.concatenate([edge_indexs[0].astype(jnp.int32), loop,
                           jnp.zeros((pad,), jnp.int32)])
    dst = jnp.concatenate([edge_indexs[1].astype(jnp.int32), loop,
                           jnp.full((pad,), N, jnp.int32)])
    sd = jnp.stack([src.reshape(NR, BLK), dst.reshape(NR, BLK)], axis=1)

    e16 = jnp.concatenate(
        [jnp.repeat(jnp.eye(H, dtype=jnp.float32), C, axis=1),
         jnp.zeros((H, D), jnp.float32)], axis=0)

    hst, asx, adx = _dense1(features, W0, _att_mat(att_src0), _att_mat(att_dst0))
    num, den = _edge_phase(hst, asx, adx, sd)

    hst2, asx2, adx2 = _dense2(num, den, e16, b0.reshape(1, D), W1,
                               _att_mat(att_src1), _att_mat(att_dst1))
    num2, den2 = _edge_phase(hst2, asx2, adx2, sd)

    return _final(num2, den2, e16, b1.reshape(1, D))


# speedup vs baseline: 40.8467x; 1.0034x over previous
import functools

import jax
import jax.numpy as jnp
from jax import lax
from jax.experimental import pallas as pl
from jax.experimental.pallas import tpu as pltpu
from jax.experimental.pallas import tpu_sc as plsc

N = 10000
D = 256
H = 8
C = D // H
E = 160000
EL = E + N
L = 16
NC = 2
NS = 16
NW = NC * NS
BLK = 128
NBT = -(-EL // (NS * BLK))
EP = NBT * NS * BLK
NBA = EP // (NW * BLK)
ABLK = 64
ANB = 2 * NBA
NR = EP // BLK
NPAD = 10240
DPAD = NPAD // 2
HD = D // NC
WR = EP // 8
BM = 2000


def _dense_tail(h, asm_ref, adm_ref, hst_ref, as_ref, ad_ref):
    z = jnp.zeros((h.shape[0], HD - L), jnp.float32)
    as2 = jnp.dot(h, asm_ref[...], preferred_element_type=jnp.float32)
    ad2 = jnp.dot(h, adm_ref[...], preferred_element_type=jnp.float32)
    hst_ref[0] = h[:, :HD]
    hst_ref[1] = h[:, HD:]
    as_ref[...] = jnp.concatenate([as2, z], axis=1)
    ad_ref[...] = jnp.concatenate([ad2, z], axis=1)


def _dense1_body(x_ref, w_ref, asm_ref, adm_ref, hst_ref, as_ref, ad_ref):
    h = jnp.dot(x_ref[...], w_ref[...], preferred_element_type=jnp.float32)
    _dense_tail(h, asm_ref, adm_ref, hst_ref, as_ref, ad_ref)


_DENSE_OUT_SPECS = [
    pl.BlockSpec((2, BM, HD), lambda i: (0, i, 0)),
    pl.BlockSpec((BM, HD), lambda i: (i, 0)),
    pl.BlockSpec((BM, HD), lambda i: (i, 0)),
]
_DENSE_OUT_SHAPE = [
    jax.ShapeDtypeStruct((2, N, HD), jnp.float32),
    jax.ShapeDtypeStruct((N, HD), jnp.float32),
    jax.ShapeDtypeStruct((N, HD), jnp.float32),
]


def _dense1(x, w, asm, adm):
    return pl.pallas_call(
        _dense1_body,
        grid=(N // BM,),
        in_specs=[
            pl.BlockSpec((BM, D), lambda i: (i, 0)),
            pl.BlockSpec((D, D), lambda i: (0, 0)),
            pl.BlockSpec((D, L), lambda i: (0, 0)),
            pl.BlockSpec((D, L), lambda i: (0, 0)),
        ],
        out_specs=_DENSE_OUT_SPECS,
        out_shape=_DENSE_OUT_SHAPE,
    )(x, w, asm, adm)


def _dense2_body(n0_ref, n1_ref, den_ref, e16_ref, b_ref, w_ref, asm_ref,
                 adm_ref, hst_ref, as_ref, ad_ref):
    dinv = 1.0 / den_ref[...]
    expand = jnp.dot(dinv, e16_ref[...], preferred_element_type=jnp.float32)
    x = jnp.concatenate([n0_ref[...], n1_ref[...]], axis=1) * expand + b_ref[...]
    h = jnp.dot(x, w_ref[...], preferred_element_type=jnp.float32)
    _dense_tail(h, asm_ref, adm_ref, hst_ref, as_ref, ad_ref)


def _dense2(num, den, e16, b, w, asm, adm):
    nb = N // BM
    return pl.pallas_call(
        _dense2_body,
        grid=(nb,),
        in_specs=[
            pl.BlockSpec((BM, HD), lambda i: (i, 0)),
            pl.BlockSpec((BM, HD), lambda i: (i + nb, 0)),
            pl.BlockSpec((BM, L), lambda i: (i, 0)),
            pl.BlockSpec((L, D), lambda i: (0, 0)),
            pl.BlockSpec((1, D), lambda i: (0, 0)),
            pl.BlockSpec((D, D), lambda i: (0, 0)),
            pl.BlockSpec((D, L), lambda i: (0, 0)),
            pl.BlockSpec((D, L), lambda i: (0, 0)),
        ],
        out_specs=_DENSE_OUT_SPECS,
        out_shape=_DENSE_OUT_SHAPE,
    )(num, num, den, e16, b, w, asm, adm)


def _final_body(n0_ref, n1_ref, den_ref, e16_ref, b_ref, o_ref):
    dinv = 1.0 / den_ref[...]
    expand = jnp.dot(dinv, e16_ref[...], preferred_element_type=jnp.float32)
    x = jnp.concatenate([n0_ref[...], n1_ref[...]], axis=1) * expand + b_ref[...]
    o_ref[...] = jax.nn.gelu(x, approximate=True)


def _final(num, den, e16, b):
    nb = N // BM
    return pl.pallas_call(
        _final_body,
        grid=(nb,),
        in_specs=[
            pl.BlockSpec((BM, HD), lambda i: (i, 0)),
            pl.BlockSpec((BM, HD), lambda i: (i + nb, 0)),
            pl.BlockSpec((BM, L), lambda i: (i, 0)),
            pl.BlockSpec((L, D), lambda i: (0, 0)),
            pl.BlockSpec((1, D), lambda i: (0, 0)),
        ],
        out_specs=pl.BlockSpec((BM, D), lambda i: (i, 0)),
        out_shape=jax.ShapeDtypeStruct((N, D), jnp.float32),
    )(num, num, den, e16, b)


_GDN = lax.GatherDimensionNumbers(
    offset_dims=(), collapsed_slice_dims=(0,), start_index_map=(0,))


def _lane_splat(v, lane):
    idx = jnp.full((L, 1), lane, jnp.int32)
    return lax.gather(v, idx, dimension_numbers=_GDN, slice_sizes=(1,),
                      mode=lax.GatherScatterMode.PROMISE_IN_BOUNDS)


_mesh = plsc.VectorSubcoreMesh(core_axis_name="c", subcore_axis_name="s")


@functools.partial(
    pl.kernel,
    out_type=(
        jax.ShapeDtypeStruct((WR, HD), jnp.float32),
        jax.ShapeDtypeStruct((2 * DPAD, HD), jnp.float32),
    ),
    mesh=_mesh,
    scratch_types=[
        pltpu.VMEM((3, BLK), jnp.int32),
        pltpu.VMEM((3, BLK), jnp.int32),
        pltpu.VMEM((1, ABLK), jnp.int32),
        pltpu.VMEM((1, ABLK), jnp.int32),
        pltpu.VMEM((ABLK, HD), jnp.float32),
        pltpu.VMEM((ABLK, HD), jnp.float32),
        pltpu.VMEM((ABLK, HD), jnp.float32),
        pltpu.VMEM((ABLK, HD), jnp.float32),
        pltpu.VMEM((ABLK, HD), jnp.float32),
        pltpu.VMEM((ABLK, HD), jnp.float32),
        pltpu.VMEM((ABLK // 8, HD), jnp.float32),
        pltpu.VMEM_SHARED((DPAD, HD), jnp.float32),
        pltpu.SemaphoreType.DMA,
        pltpu.SemaphoreType.DMA,
        pltpu.SemaphoreType.DMA,
        pltpu.SemaphoreType.DMA,
        pltpu.SemaphoreType.DMA,
        pltpu.SemaphoreType.DMA,
        pltpu.SemaphoreType.DMA,
        pltpu.SemaphoreType.DMA,
    ],
)
def _att_kernel(asx, adx, sd, w_out, den_out,
                ix0, ix1, six0, six1, asv0, asv1, adv0, adv1, den0, den1,
                wpack, den_sh, si0, si1, sa0, sa1, sb0, sb1, sd0, sd1):
    cid = lax.axis_index("c")
    sid = lax.axis_index("s")
    wid = cid * NS + sid
    ix = (ix0, ix1)
    six = (six0, six1)
    asv = (asv0, asv1)
    adv = (adv0, adv1)
    denrow = (den0, den1)
    si = (si0, si1)
    sa = (sa0, sa1)
    sb = (sb0, sb1)
    sdn = (sd0, sd1)
    rbase = wid * NBA
    wbase = wid * ANB * (ABLK // 8)

    zv = jnp.zeros((L,), jnp.float32)

    def zbody(i, carry):
        for j in range(HD // L):
            den0[i, pl.ds(j * L, L)] = zv
            den1[i, pl.ds(j * L, L)] = zv
        return carry

    lax.fori_loop(0, ABLK, zbody, 0)
    dzbase = sid * (DPAD // NS)
    for q in range(DPAD // NS // ABLK):
        pltpu.sync_copy(den0, den_sh.at[pl.ds(dzbase + q * ABLK, ABLK)])
    plsc.subcore_barrier()

    def issue_idx(r, b):
        pltpu.async_copy(sd.at[rbase + r], ix[b].at[pl.ds(0, 2)], si[b])

    def wait_idx_fix(r, b):
        pltpu.make_async_copy(sd.at[rbase + r], ix[b].at[pl.ds(0, 2)],
                              si[b]).wait()
        for j in range(BLK // L):
            s = pl.ds(j * L, L)
            ix[b][2, s] = ix[b][1, s] >> 1

    def issue_gather(ib, half, gb):
        srow = ix[ib].at[0, pl.ds(half * ABLK, ABLK)]
        drow = ix[ib].at[1, pl.ds(half * ABLK, ABLK)]
        pltpu.async_copy(asx.at[srow], asv[gb], sa[gb])
        pltpu.async_copy(adx.at[drow], adv[gb], sb[gb])

    def wait_gather(ib, half, gb):
        srow = ix[ib].at[0, pl.ds(half * ABLK, ABLK)]
        drow = ix[ib].at[1, pl.ds(half * ABLK, ABLK)]
        pltpu.make_async_copy(asx.at[srow], asv[gb], sa[gb]).wait()
        pltpu.make_async_copy(adx.at[drow], adv[gb], sb[gb]).wait()

    def wait_den_scatter(b):
        pltpu.make_async_copy(denrow[b], den_sh.at[six[b].at[0]],
                              sdn[b]).wait()

    issue_idx(0, 0)
    wait_idx_fix(0, 0)
    issue_idx(1, 1)
    issue_gather(0, 0, 0)

    def blk_body(qq, carry):
        for sub in range(4):
            blk = 4 * qq + sub
            gb = sub & 1
            half = sub & 1
            ib = (sub >> 1) & 1
            r_loc = 2 * qq + (sub >> 1)

            wait_gather(ib, half, gb)

            if half == 1:
                @pl.when(r_loc + 1 < NBA)
                def _():
                    wait_idx_fix(r_loc + 1, 1 - ib)
                    issue_gather(1 - ib, 0, 1 - gb)
            else:
                @pl.when(blk + 1 < ANB)
                def _():
                    issue_gather(ib, 1, 1 - gb)

            @pl.when(blk >= 2)
            def _():
                wait_den_scatter(gb)

            def sk(g, c2):
                dvec = ix[ib][1, pl.ds(half * ABLK + g * L, L)]
                for k2 in range(L):
                    k = g * L + k2
                    t = asv[gb][k, pl.ds(0, L)] + adv[gb][k, pl.ds(0, L)]
                    w = jnp.exp(jnp.maximum(t, 0.2 * t))
                    wpack[g * 2 + k2 // 8, pl.ds((k2 % 8) * L, L)] = w
                    dsp = _lane_splat(dvec, k2)
                    par = (dsp & 1).astype(jnp.float32)
                    denrow[gb][k, pl.ds(0, L)] = w * (1.0 - par)
                    denrow[gb][k, pl.ds(L, L)] = w * par
                return c2

            lax.fori_loop(0, ABLK // L, sk, 0)
            pltpu.sync_copy(wpack,
                            w_out.at[pl.ds(wbase + blk * (ABLK // 8),
                                           ABLK // 8)])
            for j in range(ABLK // L):
                six[gb][0, pl.ds(j * L, L)] = (
                    ix[ib][2, pl.ds(half * ABLK + j * L, L)])
            pltpu.async_copy(denrow[gb], den_sh.at[six[gb].at[0]], sdn[gb],
                             add=True)

            if half == 1:
                @pl.when(r_loc + 2 < NBA)
                def _():
                    issue_idx(r_loc + 2, ib)

        return carry

    lax.fori_loop(0, ANB // 4, blk_body, 0)
    wait_den_scatter(0)
    wait_den_scatter(1)
    plsc.subcore_barrier()

    dchunk = DPAD // NS
    pltpu.sync_copy(den_sh.at[pl.ds(sid * dchunk, dchunk)],
                    den_out.at[pl.ds(cid * DPAD + sid * dchunk, dchunk)])


@functools.partial(
    pl.kernel,
    out_type=jax.ShapeDtypeStruct((2 * N, HD), jnp.float32),
    mesh=_mesh,
    scratch_types=[
        pltpu.VMEM((2, BLK), jnp.int32),
        pltpu.VMEM((2, BLK), jnp.int32),
        pltpu.VMEM((1, BLK), jnp.int32),
        pltpu.VMEM((1, BLK), jnp.int32),
        pltpu.VMEM((BLK, HD), jnp.float32),
        pltpu.VMEM((BLK, HD), jnp.float32),
        pltpu.VMEM((BLK // 8, HD), jnp.float32),
        pltpu.VMEM((BLK // 8, HD), jnp.float32),
        pltpu.VMEM_SHARED((NPAD, HD), jnp.float32),
        pltpu.SemaphoreType.DMA,
        pltpu.SemaphoreType.DMA,
        pltpu.SemaphoreType.DMA,
        pltpu.SemaphoreType.DMA,
        pltpu.SemaphoreType.DMA,
        pltpu.SemaphoreType.DMA,
        pltpu.SemaphoreType.DMA,
        pltpu.SemaphoreType.DMA,
    ],
)
def _num_kernel(hst, w_in, sd, num_out,
                ix0, ix1, six0, six1, hv0, hv1, wv0, wv1, num_sh,
                si0, si1, sh0, sh1, sw0, sw1, ss0, ss1):
    cid = lax.axis_index("c")
    sid = lax.axis_index("s")
    ix = (ix0, ix1)
    six = (six0, six1)
    hv = (hv0, hv1)
    wv = (wv0, wv1)
    si = (si0, si1)
    sh = (sh0, sh1)
    sw = (sw0, sw1)
    ss = (ss0, ss1)
    rbase = sid * NBT
    off = cid * N

    zv = jnp.zeros((L,), jnp.float32)

    def zbody(i, carry):
        for j in range(HD // L):
            hv0[i, pl.ds(j * L, L)] = zv
        return carry

    lax.fori_loop(0, BLK, zbody, 0)
    zbase = sid * (NPAD // NS)
    for q in range(NPAD // NS // BLK):
        pltpu.sync_copy(hv0, num_sh.at[pl.ds(zbase + q * BLK, BLK)])
    plsc.subcore_barrier()

    hb = 4 * cid

    def wrows(blk):
        return w_in.at[pl.ds((rbase + blk) * (BLK // 8), BLK // 8)]

    def issue_idx(blk, b):
        pltpu.async_copy(sd.at[rbase + blk], ix[b], si[b])

    def wait_idx_fix(blk, b):
        pltpu.make_async_copy(sd.at[rbase + blk], ix[b], si[b]).wait()
        for j in range(BLK // L):
            s = pl.ds(j * L, L)
            ix[b][0, s] = ix[b][0, s] + off

    def issue_gather(blk, b):
        pltpu.async_copy(hst.at[ix[b].at[0]], hv[b], sh[b])
        pltpu.async_copy(wrows(blk), wv[b], sw[b])

    def wait_gather(blk, b):
        pltpu.make_async_copy(hst.at[ix[b].at[0]], hv[b], sh[b]).wait()
        pltpu.make_async_copy(wrows(blk), wv[b], sw[b]).wait()

    def wait_scatter(b):
        pltpu.make_async_copy(hv[b], num_sh.at[six[b].at[0]], ss[b]).wait()

    issue_idx(0, 0)
    wait_idx_fix(0, 0)
    issue_gather(0, 0)
    issue_idx(1, 1)

    def blk_body(ii, carry):
        for b in range(2):
            blk = 2 * ii + b
            wait_gather(blk, b)

            @pl.when(blk + 1 < NBT)
            def _():
                @pl.when(blk >= 1)
                def _():
                    wait_scatter(1 - b)

                wait_idx_fix(blk + 1, 1 - b)
                issue_gather(blk + 1, 1 - b)

            def sk(g, c2):
                for k2 in range(L):
                    k = g * L + k2
                    w = wv[b][g * 2 + k2 // 8, pl.ds((k2 % 8) * L, L)]
                    s = [_lane_splat(w, hb + i) for i in range(4)]
                    for j in range(HD // L):
                        sl = pl.ds(j * L, L)
                        hv[b][k, sl] = hv[b][k, sl] * s[j // 2]
                return c2

            lax.fori_loop(0, BLK // L, sk, 0)
            for j in range(BLK // L):
                s = pl.ds(j * L, L)
                six[b][0, s] = ix[b][1, s]
            pltpu.async_copy(hv[b], num_sh.at[six[b].at[0]], ss[b], add=True)

            @pl.when(blk + 2 < NBT)
            def _():
                issue_idx(blk + 2, b)

        return carry

    lax.fori_loop(0, NBT // 2, blk_body, 0)
    wait_scatter(0)
    wait_scatter(1)
    plsc.subcore_barrier()

    ochunk = 624
    obase = sid * ochunk
    pltpu.sync_copy(num_sh.at[pl.ds(obase, ochunk)],
                    num_out.at[pl.ds(off + obase, ochunk)])
    tail_base = NS * ochunk
    tail = N - tail_base

    @pl.when(sid == 0)
    def _():
        pltpu.sync_copy(num_sh.at[pl.ds(tail_base, tail)],
                        num_out.at[pl.ds(off + tail_base, tail)])


def _att_mat(a):
    eye = jnp.eye(H, dtype=jnp.float32)
    m = (eye[:, None, :] * a[:, :, None]).reshape(D, H)
    return jnp.concatenate([m, m], axis=1)


def _pad_rows(x):
    return jnp.concatenate(
        [x, jnp.zeros((NPAD - N, HD), x.dtype)], axis=0)


def _unpack_den(denp):
    d = denp.reshape(2, DPAD, HD // L, L)[:, :, :2, :].sum(0)
    return d.reshape(NPAD, L)[:N]


def _edge_phase(hst, asx, adx, sd):
    w_pk, denp = _att_kernel(_pad_rows(asx), _pad_rows(adx), sd)
    num = _num_kernel(hst.reshape(2 * N, HD), w_pk, sd)
    return num, _unpack_den(denp)


def kernel(features, edge_indexs, W0, att_src0, att_dst0, b0,
           W1, att_src1, att_dst1, b1):
    loop = jnp.arange(N, dtype=jnp.int32)
    pad = EP - EL
    src = jnp.concatenate([edge_indexs[0].astype(jnp.int32), loop,
                           jnp.zeros((pad,), jnp.int32)])
    dst = jnp.concatenate([edge_indexs[1].astype(jnp.int32), loop,
                           jnp.full((pad,), N, jnp.int32)])
    sd = jnp.stack([src.reshape(NR, BLK), dst.reshape(NR, BLK)], axis=1)

    e16 = jnp.concatenate(
        [jnp.repeat(jnp.eye(H, dtype=jnp.float32), C, axis=1),
         jnp.zeros((H, D), jnp.float32)], axis=0)

    hst, asx, adx = _dense1(features, W0, _att_mat(att_src0), _att_mat(att_dst0))
    num, den = _edge_phase(hst, asx, adx, sd)

    hst2, asx2, adx2 = _dense2(num, den, e16, b0.reshape(1, D), W1,
                               _att_mat(att_src1), _att_mat(att_dst1))
    num2, den2 = _edge_phase(hst2, asx2, adx2, sd)

    return _final(num2, den2, e16, b1.reshape(1, D))


# speedup vs baseline: 41.5390x; 1.0169x over previous
import functools

import jax
import jax.numpy as jnp
from jax import lax
from jax.experimental import pallas as pl
from jax.experimental.pallas import tpu as pltpu
from jax.experimental.pallas import tpu_sc as plsc

N = 10000
D = 256
H = 8
C = D // H
E = 160000
EL = E + N
L = 16
NC = 2
NS = 16
NW = NC * NS
BLK = 128
NBT = -(-EL // (NS * BLK))
EP = NBT * NS * BLK
NBA = EP // (NW * BLK)
ABLK = 64
ANB = 2 * NBA
NR = EP // BLK
NPAD = 10240
DPAD = NPAD // 2
HD = D // NC
WR = EP // 8
BM = 2000


def _dense_tail(h, asm_ref, adm_ref, hst_ref, as_ref, ad_ref):
    z = jnp.zeros((h.shape[0], HD - L), jnp.float32)
    as2 = jnp.dot(h, asm_ref[...], preferred_element_type=jnp.float32)
    ad2 = jnp.dot(h, adm_ref[...], preferred_element_type=jnp.float32)
    hst_ref[0] = h[:, :HD]
    hst_ref[1] = h[:, HD:]
    as_ref[...] = jnp.concatenate([as2, z], axis=1)
    ad_ref[...] = jnp.concatenate([ad2, z], axis=1)


def _dense1_body(x_ref, w_ref, asm_ref, adm_ref, hst_ref, as_ref, ad_ref):
    h = jnp.dot(x_ref[...], w_ref[...], preferred_element_type=jnp.float32)
    _dense_tail(h, asm_ref, adm_ref, hst_ref, as_ref, ad_ref)


_DENSE_OUT_SPECS = [
    pl.BlockSpec((2, BM, HD), lambda i: (0, i, 0)),
    pl.BlockSpec((BM, HD), lambda i: (i, 0)),
    pl.BlockSpec((BM, HD), lambda i: (i, 0)),
]
_DENSE_OUT_SHAPE = [
    jax.ShapeDtypeStruct((2, N, HD), jnp.float32),
    jax.ShapeDtypeStruct((N, HD), jnp.float32),
    jax.ShapeDtypeStruct((N, HD), jnp.float32),
]


def _dense1(x, w, asm, adm):
    return pl.pallas_call(
        _dense1_body,
        grid=(N // BM,),
        in_specs=[
            pl.BlockSpec((BM, D), lambda i: (i, 0)),
            pl.BlockSpec((D, D), lambda i: (0, 0)),
            pl.BlockSpec((D, L), lambda i: (0, 0)),
            pl.BlockSpec((D, L), lambda i: (0, 0)),
        ],
        out_specs=_DENSE_OUT_SPECS,
        out_shape=_DENSE_OUT_SHAPE,
    )(x, w, asm, adm)


def _dense2_body(n0_ref, n1_ref, den_ref, e16_ref, b_ref, w_ref, asm_ref,
                 adm_ref, hst_ref, as_ref, ad_ref):
    dinv = 1.0 / den_ref[...]
    expand = jnp.dot(dinv, e16_ref[...], preferred_element_type=jnp.float32)
    x = jnp.concatenate([n0_ref[...], n1_ref[...]], axis=1) * expand + b_ref[...]
    h = jnp.dot(x, w_ref[...], preferred_element_type=jnp.float32)
    _dense_tail(h, asm_ref, adm_ref, hst_ref, as_ref, ad_ref)


def _dense2(num, den, e16, b, w, asm, adm):
    nb = N // BM
    return pl.pallas_call(
        _dense2_body,
        grid=(nb,),
        in_specs=[
            pl.BlockSpec((BM, HD), lambda i: (i, 0)),
            pl.BlockSpec((BM, HD), lambda i: (i + nb, 0)),
            pl.BlockSpec((BM, L), lambda i: (i, 0)),
            pl.BlockSpec((L, D), lambda i: (0, 0)),
            pl.BlockSpec((1, D), lambda i: (0, 0)),
            pl.BlockSpec((D, D), lambda i: (0, 0)),
            pl.BlockSpec((D, L), lambda i: (0, 0)),
            pl.BlockSpec((D, L), lambda i: (0, 0)),
        ],
        out_specs=_DENSE_OUT_SPECS,
        out_shape=_DENSE_OUT_SHAPE,
    )(num, num, den, e16, b, w, asm, adm)


def _final_body(n0_ref, n1_ref, den_ref, e16_ref, b_ref, o_ref):
    dinv = 1.0 / den_ref[...]
    expand = jnp.dot(dinv, e16_ref[...], preferred_element_type=jnp.float32)
    x = jnp.concatenate([n0_ref[...], n1_ref[...]], axis=1) * expand + b_ref[...]
    o_ref[...] = jax.nn.gelu(x, approximate=True)


def _final(num, den, e16, b):
    nb = N // BM
    return pl.pallas_call(
        _final_body,
        grid=(nb,),
        in_specs=[
            pl.BlockSpec((BM, HD), lambda i: (i, 0)),
            pl.BlockSpec((BM, HD), lambda i: (i + nb, 0)),
            pl.BlockSpec((BM, L), lambda i: (i, 0)),
            pl.BlockSpec((L, D), lambda i: (0, 0)),
            pl.BlockSpec((1, D), lambda i: (0, 0)),
        ],
        out_specs=pl.BlockSpec((BM, D), lambda i: (i, 0)),
        out_shape=jax.ShapeDtypeStruct((N, D), jnp.float32),
    )(num, num, den, e16, b)


_GDN = lax.GatherDimensionNumbers(
    offset_dims=(), collapsed_slice_dims=(0,), start_index_map=(0,))


def _lane_splat(v, lane):
    idx = jnp.full((L, 1), lane, jnp.int32)
    return lax.gather(v, idx, dimension_numbers=_GDN, slice_sizes=(1,),
                      mode=lax.GatherScatterMode.PROMISE_IN_BOUNDS)


_mesh = plsc.VectorSubcoreMesh(core_axis_name="c", subcore_axis_name="s")


@functools.partial(
    pl.kernel,
    out_type=(
        jax.ShapeDtypeStruct((WR, HD), jnp.float32),
        jax.ShapeDtypeStruct((2 * DPAD, HD), jnp.float32),
    ),
    mesh=_mesh,
    scratch_types=[
        pltpu.VMEM((3, BLK), jnp.int32),
        pltpu.VMEM((3, BLK), jnp.int32),
        pltpu.VMEM((1, ABLK), jnp.int32),
        pltpu.VMEM((1, ABLK), jnp.int32),
        pltpu.VMEM((ABLK, HD), jnp.float32),
        pltpu.VMEM((ABLK, HD), jnp.float32),
        pltpu.VMEM((ABLK, HD), jnp.float32),
        pltpu.VMEM((ABLK, HD), jnp.float32),
        pltpu.VMEM((ABLK, HD), jnp.float32),
        pltpu.VMEM((ABLK, HD), jnp.float32),
        pltpu.VMEM((ABLK // 8, HD), jnp.float32),
        pltpu.VMEM((ABLK // 8, HD), jnp.float32),
        pltpu.VMEM_SHARED((DPAD, HD), jnp.float32),
        pltpu.SemaphoreType.DMA,
        pltpu.SemaphoreType.DMA,
        pltpu.SemaphoreType.DMA,
        pltpu.SemaphoreType.DMA,
        pltpu.SemaphoreType.DMA,
        pltpu.SemaphoreType.DMA,
        pltpu.SemaphoreType.DMA,
        pltpu.SemaphoreType.DMA,
        pltpu.SemaphoreType.DMA,
        pltpu.SemaphoreType.DMA,
    ],
)
def _att_kernel(asx, adx, sd, w_out, den_out,
                ix0, ix1, six0, six1, asv0, asv1, adv0, adv1, den0, den1,
                wp0, wp1, den_sh, si0, si1, sa0, sa1, sb0, sb1, sd0, sd1,
                sw0, sw1):
    cid = lax.axis_index("c")
    sid = lax.axis_index("s")
    wid = cid * NS + sid
    ix = (ix0, ix1)
    six = (six0, six1)
    asv = (asv0, asv1)
    adv = (adv0, adv1)
    denrow = (den0, den1)
    si = (si0, si1)
    sa = (sa0, sa1)
    sb = (sb0, sb1)
    sdn = (sd0, sd1)
    wpack = (wp0, wp1)
    swp = (sw0, sw1)
    rbase = wid * NBA
    wbase = wid * ANB * (ABLK // 8)

    zv = jnp.zeros((L,), jnp.float32)

    def issue_idx(r, b):
        pltpu.async_copy(sd.at[rbase + r], ix[b].at[pl.ds(0, 2)], si[b])

    def wait_idx_fix(r, b):
        pltpu.make_async_copy(sd.at[rbase + r], ix[b].at[pl.ds(0, 2)],
                              si[b]).wait()
        for j in range(BLK // L):
            s = pl.ds(j * L, L)
            ix[b][2, s] = ix[b][1, s] >> 1

    def issue_gather(ib, half, gb):
        srow = ix[ib].at[0, pl.ds(half * ABLK, ABLK)]
        drow = ix[ib].at[1, pl.ds(half * ABLK, ABLK)]
        pltpu.async_copy(asx.at[srow], asv[gb], sa[gb])
        pltpu.async_copy(adx.at[drow], adv[gb], sb[gb])

    def wait_gather(ib, half, gb):
        srow = ix[ib].at[0, pl.ds(half * ABLK, ABLK)]
        drow = ix[ib].at[1, pl.ds(half * ABLK, ABLK)]
        pltpu.make_async_copy(asx.at[srow], asv[gb], sa[gb]).wait()
        pltpu.make_async_copy(adx.at[drow], adv[gb], sb[gb]).wait()

    def wait_den_scatter(b):
        pltpu.make_async_copy(denrow[b], den_sh.at[six[b].at[0]],
                              sdn[b]).wait()

    def wait_w_write(blk, b):
        pltpu.make_async_copy(
            wpack[b], w_out.at[pl.ds(wbase + blk * (ABLK // 8), ABLK // 8)],
            swp[b]).wait()

    issue_idx(0, 0)
    wait_idx_fix(0, 0)
    issue_idx(1, 1)
    issue_gather(0, 0, 0)

    def zbody(i, carry):
        for j in range(HD // L):
            den0[i, pl.ds(j * L, L)] = zv
            den1[i, pl.ds(j * L, L)] = zv
        return carry

    lax.fori_loop(0, ABLK, zbody, 0)
    dzbase = sid * (DPAD // NS)
    for q in range(DPAD // NS // ABLK):
        pltpu.sync_copy(den0, den_sh.at[pl.ds(dzbase + q * ABLK, ABLK)])
    plsc.subcore_barrier()

    def blk_body(qq, carry):
        for sub in range(4):
            blk = 4 * qq + sub
            gb = sub & 1
            half = sub & 1
            ib = (sub >> 1) & 1
            r_loc = 2 * qq + (sub >> 1)

            wait_gather(ib, half, gb)

            if half == 1:
                @pl.when(r_loc + 1 < NBA)
                def _():
                    wait_idx_fix(r_loc + 1, 1 - ib)
                    issue_gather(1 - ib, 0, 1 - gb)
            else:
                @pl.when(blk + 1 < ANB)
                def _():
                    issue_gather(ib, 1, 1 - gb)

            @pl.when(blk >= 2)
            def _():
                wait_den_scatter(gb)
                wait_w_write(blk - 2, gb)

            def sk(g, c2):
                dvec = ix[ib][1, pl.ds(half * ABLK + g * L, L)]
                for k2 in range(L):
                    k = g * L + k2
                    t = asv[gb][k, pl.ds(0, L)] + adv[gb][k, pl.ds(0, L)]
                    w = jnp.exp(jnp.maximum(t, 0.2 * t))
                    wpack[gb][g * 2 + k2 // 8, pl.ds((k2 % 8) * L, L)] = w
                    dsp = _lane_splat(dvec, k2)
                    par = (dsp & 1).astype(jnp.float32)
                    denrow[gb][k, pl.ds(0, L)] = w * (1.0 - par)
                    denrow[gb][k, pl.ds(L, L)] = w * par
                return c2

            lax.fori_loop(0, ABLK // L, sk, 0)
            pltpu.async_copy(wpack[gb],
                             w_out.at[pl.ds(wbase + blk * (ABLK // 8),
                                            ABLK // 8)], swp[gb])
            for j in range(ABLK // L):
                six[gb][0, pl.ds(j * L, L)] = (
                    ix[ib][2, pl.ds(half * ABLK + j * L, L)])
            pltpu.async_copy(denrow[gb], den_sh.at[six[gb].at[0]], sdn[gb],
                             add=True)

            if half == 1:
                @pl.when(r_loc + 2 < NBA)
                def _():
                    issue_idx(r_loc + 2, ib)

        return carry

    lax.fori_loop(0, ANB // 4, blk_body, 0)
    wait_den_scatter(0)
    wait_den_scatter(1)
    wait_w_write(ANB - 2, 0)
    wait_w_write(ANB - 1, 1)
    plsc.subcore_barrier()

    dchunk = DPAD // NS
    pltpu.sync_copy(den_sh.at[pl.ds(sid * dchunk, dchunk)],
                    den_out.at[pl.ds(cid * DPAD + sid * dchunk, dchunk)])


@functools.partial(
    pl.kernel,
    out_type=jax.ShapeDtypeStruct((2 * N, HD), jnp.float32),
    mesh=_mesh,
    scratch_types=[
        pltpu.VMEM((2, BLK), jnp.int32),
        pltpu.VMEM((2, BLK), jnp.int32),
        pltpu.VMEM((1, BLK), jnp.int32),
        pltpu.VMEM((1, BLK), jnp.int32),
        pltpu.VMEM((BLK, HD), jnp.float32),
        pltpu.VMEM((BLK, HD), jnp.float32),
        pltpu.VMEM((BLK // 8, HD), jnp.float32),
        pltpu.VMEM((BLK // 8, HD), jnp.float32),
        pltpu.VMEM_SHARED((NPAD, HD), jnp.float32),
        pltpu.SemaphoreType.DMA,
        pltpu.SemaphoreType.DMA,
        pltpu.SemaphoreType.DMA,
        pltpu.SemaphoreType.DMA,
        pltpu.SemaphoreType.DMA,
        pltpu.SemaphoreType.DMA,
        pltpu.SemaphoreType.DMA,
        pltpu.SemaphoreType.DMA,
    ],
)
def _num_kernel(hst, w_in, sd, num_out,
                ix0, ix1, six0, six1, hv0, hv1, wv0, wv1, num_sh,
                si0, si1, sh0, sh1, sw0, sw1, ss0, ss1):
    cid = lax.axis_index("c")
    sid = lax.axis_index("s")
    ix = (ix0, ix1)
    six = (six0, six1)
    hv = (hv0, hv1)
    wv = (wv0, wv1)
    si = (si0, si1)
    sh = (sh0, sh1)
    sw = (sw0, sw1)
    ss = (ss0, ss1)
    rbase = sid * NBT
    off = cid * N

    zv = jnp.zeros((L,), jnp.float32)
    hb = 4 * cid

    def wrows(blk):
        return w_in.at[pl.ds((rbase + blk) * (BLK // 8), BLK // 8)]

    def issue_idx(blk, b):
        pltpu.async_copy(sd.at[rbase + blk], ix[b], si[b])

    def wait_idx_fix(blk, b):
        pltpu.make_async_copy(sd.at[rbase + blk], ix[b], si[b]).wait()
        for j in range(BLK // L):
            s = pl.ds(j * L, L)
            ix[b][0, s] = ix[b][0, s] + off

    def issue_gather(blk, b):
        pltpu.async_copy(hst.at[ix[b].at[0]], hv[b], sh[b])
        pltpu.async_copy(wrows(blk), wv[b], sw[b])

    def wait_gather(blk, b):
        pltpu.make_async_copy(hst.at[ix[b].at[0]], hv[b], sh[b]).wait()
        pltpu.make_async_copy(wrows(blk), wv[b], sw[b]).wait()

    def wait_scatter(b):
        pltpu.make_async_copy(hv[b], num_sh.at[six[b].at[0]], ss[b]).wait()

    issue_idx(0, 0)
    wait_idx_fix(0, 0)
    issue_gather(0, 0)
    issue_idx(1, 1)

    def zbody(i, carry):
        for j in range(HD // L):
            hv1[i, pl.ds(j * L, L)] = zv
        return carry

    lax.fori_loop(0, BLK, zbody, 0)
    zbase = sid * (NPAD // NS)
    for q in range(NPAD // NS // BLK):
        pltpu.sync_copy(hv1, num_sh.at[pl.ds(zbase + q * BLK, BLK)])
    plsc.subcore_barrier()

    def blk_body(ii, carry):
        for b in range(2):
            blk = 2 * ii + b
            wait_gather(blk, b)

            @pl.when(blk + 1 < NBT)
            def _():
                @pl.when(blk >= 1)
                def _():
                    wait_scatter(1 - b)

                wait_idx_fix(blk + 1, 1 - b)
                issue_gather(blk + 1, 1 - b)

            def sk(g, c2):
                for k2 in range(L):
                    k = g * L + k2
                    w = wv[b][g * 2 + k2 // 8, pl.ds((k2 % 8) * L, L)]
                    s = [_lane_splat(w, hb + i) for i in range(4)]
                    for j in range(HD // L):
                        sl = pl.ds(j * L, L)
                        hv[b][k, sl] = hv[b][k, sl] * s[j // 2]
                return c2

            lax.fori_loop(0, BLK // L, sk, 0)
            for j in range(BLK // L):
                s = pl.ds(j * L, L)
                six[b][0, s] = ix[b][1, s]
            pltpu.async_copy(hv[b], num_sh.at[six[b].at[0]], ss[b], add=True)

            @pl.when(blk + 2 < NBT)
            def _():
                issue_idx(blk + 2, b)

        return carry

    lax.fori_loop(0, NBT // 2, blk_body, 0)
    wait_scatter(0)
    wait_scatter(1)
    plsc.subcore_barrier()

    ochunk = 624
    obase = sid * ochunk
    pltpu.sync_copy(num_sh.at[pl.ds(obase, ochunk)],
                    num_out.at[pl.ds(off + obase, ochunk)])
    tail_base = NS * ochunk
    tail = N - tail_base

    @pl.when(sid == 0)
    def _():
        pltpu.sync_copy(num_sh.at[pl.ds(tail_base, tail)],
                        num_out.at[pl.ds(off + tail_base, tail)])


def _att_mat(a):
    eye = jnp.eye(H, dtype=jnp.float32)
    m = (eye[:, None, :] * a[:, :, None]).reshape(D, H)
    return jnp.concatenate([m, m], axis=1)


def _pad_rows(x):
    return jnp.concatenate(
        [x, jnp.zeros((NPAD - N, HD), x.dtype)], axis=0)


def _unpack_den(denp):
    d = denp.reshape(2, DPAD, HD // L, L)[:, :, :2, :].sum(0)
    return d.reshape(NPAD, L)[:N]


def _edge_phase(hst, asx, adx, sd):
    w_pk, denp = _att_kernel(_pad_rows(asx), _pad_rows(adx), sd)
    num = _num_kernel(hst.reshape(2 * N, HD), w_pk, sd)
    return num, _unpack_den(denp)


def kernel(features, edge_indexs, W0, att_src0, att_dst0, b0,
           W1, att_src1, att_dst1, b1):
    loop = jnp.arange(N, dtype=jnp.int32)
    pad = EP - EL
    src = jnp.concatenate([edge_indexs[0].astype(jnp.int32), loop,
                           jnp.zeros((pad,), jnp.int32)])
    dst = jnp.concatenate([edge_indexs[1].astype(jnp.int32), loop,
                           jnp.full((pad,), N, jnp.int32)])
    sd = jnp.stack([src.reshape(NR, BLK), dst.reshape(NR, BLK)], axis=1)

    e16 = jnp.concatenate(
        [jnp.repeat(jnp.eye(H, dtype=jnp.float32), C, axis=1),
         jnp.zeros((H, D), jnp.float32)], axis=0)

    hst, asx, adx = _dense1(features, W0, _att_mat(att_src0), _att_mat(att_dst0))
    num, den = _edge_phase(hst, asx, adx, sd)

    hst2, asx2, adx2 = _dense2(num, den, e16, b0.reshape(1, D), W1,
                               _att_mat(att_src1), _att_mat(att_dst1))
    num2, den2 = _edge_phase(hst2, asx2, adx2, sd)

    return _final(num2, den2, e16, b1.reshape(1, D))
